# Initial kernel scaffold; baseline (speedup 1.0000x reference)
import jax
import jax.numpy as jnp
from jax.experimental import pallas as pl


def kernel(xyz, params):
    raise NotImplementedError("write your pallas kernel here")


# speedup vs baseline: 5.3502x; 5.3502x over previous
import functools

import jax
import jax.numpy as jnp
from jax import lax
from jax.experimental import pallas as pl
from jax.experimental.pallas import tpu as pltpu
from jax.experimental.pallas import tpu_sc as plsc

_EPS = 1e-5
_B, _N = 4, 1024
_INTERPRET = False


def _fps_body(x_ref, y_ref, z_ref, nx_ref, ny_ref, nz_ref, *, npoint):
    X = x_ref[...]
    Y = y_ref[...]
    Z = z_ref[...]
    b, n = X.shape
    col = lax.broadcasted_iota(jnp.int32, (b, n), 1)
    col_s = lax.broadcasted_iota(jnp.int32, (b, npoint), 1)

    def step(t, carry):
        dist, far, ax, ay, az = carry
        sel = col == far
        cx = jnp.sum(jnp.where(sel, X, 0.0), axis=1, keepdims=True)
        cy = jnp.sum(jnp.where(sel, Y, 0.0), axis=1, keepdims=True)
        cz = jnp.sum(jnp.where(sel, Z, 0.0), axis=1, keepdims=True)
        hit = col_s == t
        ax = jnp.where(hit, cx, ax)
        ay = jnp.where(hit, cy, ay)
        az = jnp.where(hit, cz, az)
        dx = X - cx
        dy = Y - cy
        dz = Z - cz
        d = dx * dx + dy * dy + dz * dz
        dist = jnp.minimum(dist, d)
        mx = jnp.max(dist, axis=1, keepdims=True)
        far_new = jnp.min(
            jnp.where(dist == mx, col, jnp.int32(n)), axis=1, keepdims=True
        )
        return dist, far_new, ax, ay, az

    init = (
        jnp.full((b, n), 1e10, jnp.float32),
        jnp.zeros((b, 1), jnp.int32),
        jnp.zeros((b, npoint), jnp.float32),
        jnp.zeros((b, npoint), jnp.float32),
        jnp.zeros((b, npoint), jnp.float32),
    )
    _, _, ax, ay, az = lax.fori_loop(0, npoint, step, init)
    nx_ref[...] = ax
    ny_ref[...] = ay
    nz_ref[...] = az


def _fps(X, Y, Z, npoint):
    b, n = X.shape
    out = jax.ShapeDtypeStruct((b, npoint), jnp.float32)
    nx, ny, nz = pl.pallas_call(
        functools.partial(_fps_body, npoint=npoint),
        out_shape=(out, out, out),
        interpret=_INTERPRET,
    )(X, Y, Z)
    return nx, ny, nz


def _mask_body(new_ref, ptsT_ref, o_ref, *, r2s):
    new = new_ref[0]
    ptsT = ptsT_ref[0]
    pn2 = jnp.sum(ptsT * ptsT, axis=0, keepdims=True)
    cn2 = jnp.sum(new * new, axis=1, keepdims=True)
    d = (cn2 + pn2) - 2.0 * jnp.dot(new, ptsT,
                                    preferred_element_type=jnp.float32)
    w = jnp.zeros(d.shape, jnp.int32)
    for i, r2 in enumerate(r2s):
        w = w + jnp.where(d <= jnp.float32(r2), jnp.int32(1 << i),
                          jnp.int32(0))
    o_ref[0] = w


def _masks(new_xyz, ptsT, r2s):
    b, s, _ = new_xyz.shape
    n = ptsT.shape[2]
    out = pl.pallas_call(
        functools.partial(_mask_body, r2s=tuple(r2s)),
        grid=(b,),
        in_specs=[
            pl.BlockSpec((1, s, 3), lambda i: (i, 0, 0)),
            pl.BlockSpec((1, 3, n), lambda i: (i, 0, 0)),
        ],
        out_specs=pl.BlockSpec((1, s, n), lambda i: (i, 0, 0)),
        out_shape=jax.ShapeDtypeStruct((b, s, n), jnp.int32),
        interpret=_INTERPRET,
    )(new_xyz, ptsT)
    return out.reshape(b * s, n)


def _table_body(*refs, nw):
    pts = refs[0][...]
    idx = 1
    xyzv = None
    if nw == 2:
        xyzv = refs[1][...]
        idx = 2
    n_br = (len(refs) - idx) // (nw + 2)
    ins = refs[idx:]
    outs = refs[idx + n_br * (nw + 1):]
    for br in range(n_br):
        wp = ins[br * (nw + 1)][...]
        t = jnp.dot(pts, wp, preferred_element_type=jnp.float32)
        if nw == 2:
            wx = ins[br * (nw + 1) + 1][...]
            t = t + jnp.dot(xyzv, wx, preferred_element_type=jnp.float32)
        bias = ins[br * (nw + 1) + nw][...]
        outs[br][...] = t + bias


def _tables(pts, xyz, weights):
    nw = 2 if xyz is not None else 1
    ins = [pts] + ([xyz] if xyz is not None else [])
    for wset in weights:
        ins.extend(wset)
    outs = tuple(
        jax.ShapeDtypeStruct((pts.shape[0], w[0].shape[1]), jnp.float32)
        for w in weights
    )
    return pl.pallas_call(
        functools.partial(_table_body, nw=nw),
        out_shape=outs,
        interpret=_INTERPRET,
    )(*ins)


@functools.lru_cache(maxsize=None)
def _sc_group_builder(BS, N, S, Ks, Cs):
    info = plsc.get_sparse_core_info()
    NC, NS = info.num_cores, info.num_subcores
    NW = NC * NS
    per_w = BS // NW
    nch = N // 16
    BN = (BS // S) * N

    out_type = tuple(
        jax.ShapeDtypeStruct((BS * K, C), jnp.float32)
        for K, C in zip(Ks, Cs)
    )
    scratch = [pltpu.VMEM((N,), jnp.int32)]
    for K, C in zip(Ks, Cs):
        scratch += [
            pltpu.VMEM((N + 16,), jnp.int32),
            pltpu.VMEM((K,), jnp.int32),
            pltpu.VMEM((K, C), jnp.float32),
        ]
    scratch.append(pltpu.SMEM((8,), jnp.int32))
    scratch.append(pltpu.SemaphoreType.DMA)
    mesh = plsc.VectorSubcoreMesh(core_axis_name="c", subcore_axis_name="s")

    @functools.partial(pl.kernel, mesh=mesh, out_type=out_type,
                       scratch_types=scratch,
                       compiler_params=pltpu.CompilerParams(
                           needs_layout_passes=False))
    def sc_kernel(mask_hbm, t0, t1, t2, g0, g1, g2,
                  mw, ib0, ix0, rw0, ib1, ix1, rw1, ib2, ix2, rw2, cnt, sem):
        T = (t0, t1, t2)
        G = (g0, g1, g2)
        IB = (ib0, ib1, ib2)
        IX = (ix0, ix1, ix2)
        RW = (rw0, rw1, rw2)
        wid = lax.axis_index("s") * NC + lax.axis_index("c")
        lane = lax.iota(jnp.int32, 16)
        zeros16 = jnp.zeros((16,), jnp.int32)
        ones16 = jnp.full((16,), 1, jnp.int32)

        def splat(x):
            return jnp.full((16,), x, jnp.int32)

        pmax16 = jnp.full((16,), N + 15, jnp.int32)
        imax16 = jnp.full((16,), BN - 1, jnp.int32)

        def per_centroid(ci, _):
            sg = wid * per_w + ci
            bidx = sg // S
            base = bidx * N
            pltpu.sync_copy(mask_hbm.at[pl.ds(sg * N, N)], mw)
            for br in range(3):
                cnt[br] = jnp.int32(0)

            def chunk(ch, _):
                w = mw[pl.ds(ch * 16, 16)]
                iv = lane + splat(ch * 16)
                for br in range(3):
                    c = cnt[br]
                    m = ((w >> splat(br)) & ones16) == ones16
                    mi = m.astype(jnp.int32)
                    pos = plsc.cumsum(mi) + splat(c - 1)
                    pos = jnp.minimum(jnp.maximum(pos, zeros16), pmax16)
                    plsc.store_scatter(IB[br], [pos], iv, mask=m)
                    cnt[br] = c + jnp.sum(mi)
                return 0

            lax.fori_loop(0, nch, chunk, 0)
            for br in range(3):
                K = Ks[br]
                c0 = IB[br][pl.ds(0, 16)]
                first = splat(jnp.sum(jnp.where(lane == zeros16, c0,
                                                zeros16)))
                cb = cnt[br]
                first = jnp.where(splat(cb) == zeros16,
                                  jnp.full((16,), N - 1, jnp.int32), first)
                for j in range(K // 16):
                    li = lane + splat(j * 16)
                    v = IB[br][pl.ds(j * 16, 16)]
                    v = jnp.where(li < splat(cb), v, first) + splat(base)
                    v = jnp.minimum(jnp.maximum(v, zeros16), imax16)
                    IX[br][pl.ds(j * 16, 16)] = v
                pltpu.async_copy(T[br].at[IX[br]], RW[br], sem).wait()
                pltpu.sync_copy(RW[br], G[br].at[pl.ds(sg * K, K)])
            return 0

        lax.fori_loop(0, per_w, per_centroid, 0)

    return sc_kernel


def _sc_group(mask, tables, BS, N, S, Ks, Cs):
    k = _sc_group_builder(BS, N, S, tuple(Ks), tuple(Cs))
    return k(mask.reshape(BS * N), *tables)


def _branch_body(g_ref, nxyz_ref, wx_ref, w2_ref, b2_ref, w3_ref, b3_ref,
                 o_ref, *, SB, K):
    D = jnp.dot(nxyz_ref[...], wx_ref[...],
                preferred_element_type=jnp.float32)
    G = g_ref[...]
    C1 = G.shape[1]
    h = jnp.maximum(G.reshape(SB, K, C1) - D[:, None, :], 0.0)
    h = h.reshape(SB * K, C1)
    h = jnp.maximum(
        jnp.dot(h, w2_ref[...], preferred_element_type=jnp.float32)
        + b2_ref[...], 0.0)
    h = jnp.maximum(
        jnp.dot(h, w3_ref[...], preferred_element_type=jnp.float32)
        + b3_ref[...], 0.0)
    C3 = h.shape[1]
    o_ref[...] = jnp.max(h.reshape(SB, K, C3), axis=1)


def _branch(G, nxyz, wx, w2, b2, w3, b3, K, SB):
    BS = nxyz.shape[0]
    C1 = wx.shape[1]
    C3 = w3.shape[1]
    grid = (BS // SB,)
    return pl.pallas_call(
        functools.partial(_branch_body, SB=SB, K=K),
        grid=grid,
        in_specs=[
            pl.BlockSpec((SB * K, C1), lambda i: (i, 0)),
            pl.BlockSpec((SB, 3), lambda i: (i, 0)),
            pl.BlockSpec(wx.shape, lambda i: (0, 0)),
            pl.BlockSpec(w2.shape, lambda i: (0, 0)),
            pl.BlockSpec(b2.shape, lambda i: (0, 0)),
            pl.BlockSpec(w3.shape, lambda i: (0, 0)),
            pl.BlockSpec(b3.shape, lambda i: (0, 0)),
        ],
        out_specs=pl.BlockSpec((SB, C3), lambda i: (i, 0)),
        out_shape=jax.ShapeDtypeStruct((BS, C3), jnp.float32),
        interpret=_INTERPRET,
    )(G, nxyz, wx, w2, b2, w3, b3)


def _final_body(xyz_ref, pts_ref, wx_ref, wp_ref, b1_ref, w2_ref, b2_ref,
                w3_ref, b3_ref, wf1_ref, bf1_ref, wf2_ref, bf2_ref,
                wf3_ref, bf3_ref, o_ref, *, b, s):
    h = jnp.dot(xyz_ref[...], wx_ref[...],
                preferred_element_type=jnp.float32)
    h = h + jnp.dot(pts_ref[...], wp_ref[...],
                    preferred_element_type=jnp.float32)
    h = jnp.maximum(h + b1_ref[...], 0.0)
    h = jnp.maximum(
        jnp.dot(h, w2_ref[...], preferred_element_type=jnp.float32)
        + b2_ref[...], 0.0)
    h = jnp.maximum(
        jnp.dot(h, w3_ref[...], preferred_element_type=jnp.float32)
        + b3_ref[...], 0.0)
    C = h.shape[1]
    hm = jnp.max(h.reshape(b, s, C), axis=1)
    f = jnp.maximum(
        jnp.dot(hm, wf1_ref[...], preferred_element_type=jnp.float32)
        + bf1_ref[...], 0.0)
    f = jnp.maximum(
        jnp.dot(f, wf2_ref[...], preferred_element_type=jnp.float32)
        + bf2_ref[...], 0.0)
    o_ref[...] = (
        jnp.dot(f, wf3_ref[...], preferred_element_type=jnp.float32)
        + bf3_ref[...])


def _final(xyz2, pts2, ws, b, s, ncls):
    return pl.pallas_call(
        functools.partial(_final_body, b=b, s=s),
        out_shape=jax.ShapeDtypeStruct((b, ncls), jnp.float32),
        interpret=_INTERPRET,
    )(xyz2, pts2, *ws)


def _fold(layer):
    w, bb, gamma, beta = layer
    s = gamma / jnp.sqrt(1.0 + _EPS)
    return (w * s[:, None]).T, (s * bb + beta)[None, :]


def _fold_split(layer, n_head):
    wf, bias = _fold(layer)
    return wf[:n_head], wf[n_head:], bias


_CP = 128


def _pad_cols(a):
    return jnp.pad(a, ((0, 0), (0, _CP - a.shape[1])))


_SA1 = dict(S=512, radii=(0.1, 0.2, 0.4), Ks=(16, 32, 128), SBs=(128, 128, 64))
_SA2 = dict(S=128, radii=(0.2, 0.4, 0.8), Ks=(32, 64, 128), SBs=(128, 64, 32))


def _sa_msg(pts_feat, X, Y, Z, cfg, branch_params, b, n):
    s = cfg["S"]
    nx, ny, nz = _fps(X, Y, Z, s)
    new_xyz = jnp.stack([nx, ny, nz], axis=-1)
    ptsT = jnp.stack([X, Y, Z], axis=1)
    mask = _masks(new_xyz, ptsT, [r * r for r in cfg["radii"]])

    xyz_flat = jnp.stack([X, Y, Z], axis=-1).reshape(b * n, 3)
    wsets = []
    folded = []
    for layers in branch_params:
        if pts_feat is None:
            wx, bias = _fold(layers[0])
            wx = _pad_cols(wx)
            bias = _pad_cols(bias)
            wsets.append((wx, bias))
            rest = [_fold(l) for l in layers[1:]]
        else:
            cin = pts_feat.shape[2]
            wp, wx, bias = _fold_split(layers[0], cin)
            wp, wx, bias = _pad_cols(wp), _pad_cols(wx), _pad_cols(bias)
            wsets.append((wp, wx, bias))
            rest = [_fold(l) for l in layers[1:]]
        (w2, b2), (w3, b3) = rest
        w2 = jnp.pad(w2, ((0, _CP - w2.shape[0]), (0, 0)))
        folded.append((wx, (w2, b2), (w3, b3)))

    if pts_feat is None:
        tabs = _tables(xyz_flat, None, wsets)
    else:
        tabs = _tables(pts_feat.reshape(b * n, -1), xyz_flat, wsets)

    Cs = tuple(t.shape[1] for t in tabs)
    Gs = _sc_group(mask, tabs, b * s, n, s, cfg["Ks"], Cs)

    new_flat = new_xyz.reshape(b * s, 3)
    outs = []
    for br in range(3):
        wx = folded[br][0]
        (w2, b2), (w3, b3) = folded[br][1], folded[br][2]
        o = _branch(Gs[br], new_flat, wx, w2, b2, w3, b3,
                    cfg["Ks"][br], cfg["SBs"][br])
        outs.append(o.reshape(b, s, -1))
    return new_xyz, jnp.concatenate(outs, axis=-1)


def kernel(xyz, params):
    b = xyz.shape[0]
    X, Y, Z = xyz[:, 0, :], xyz[:, 1, :], xyz[:, 2, :]

    l1_xyz, l1_pts = _sa_msg(None, X, Y, Z, _SA1, params["sa1"], b, _N)
    X2, Y2, Z2 = l1_xyz[..., 0], l1_xyz[..., 1], l1_xyz[..., 2]
    l2_xyz, l2_pts = _sa_msg(l1_pts, X2, Y2, Z2, _SA2, params["sa2"],
                             b, _SA1["S"])

    s2 = _SA2["S"]
    wx3, wp3, b31 = _fold_split(params["sa3"][0], 3)
    (w32, b32), (w33, b33) = (_fold(params["sa3"][1]),
                              _fold(params["sa3"][2]))
    wf1, bf1 = _fold(params["fc1"])
    wf2, bf2 = _fold(params["fc2"])
    w3, b3 = params["fc3"]
    ws = (wx3, wp3, b31, w32, b32, w33, b33,
          wf1, bf1, wf2, bf2, w3.T, b3[None, :])
    return _final(l2_xyz.reshape(b * s2, 3), l2_pts.reshape(b * s2, -1),
                  ws, b, s2, w3.shape[0])


# speedup vs baseline: 9.0260x; 1.6870x over previous
import functools

import jax
import jax.numpy as jnp
from jax import lax
from jax.experimental import pallas as pl
from jax.experimental.pallas import tpu as pltpu
from jax.experimental.pallas import tpu_sc as plsc

_EPS = 1e-5
_B, _N = 4, 1024
_INTERPRET = False


def _fps_body(x_ref, y_ref, z_ref, nx_ref, ny_ref, nz_ref, *, npoint):
    X = x_ref[...]
    Y = y_ref[...]
    Z = z_ref[...]
    b, n = X.shape
    col = lax.broadcasted_iota(jnp.int32, (b, n), 1)
    col_s = lax.broadcasted_iota(jnp.int32, (b, npoint), 1)

    def step(t, carry):
        dist, far, ax, ay, az = carry
        sel = col == far
        cx = jnp.sum(jnp.where(sel, X, 0.0), axis=1, keepdims=True)
        cy = jnp.sum(jnp.where(sel, Y, 0.0), axis=1, keepdims=True)
        cz = jnp.sum(jnp.where(sel, Z, 0.0), axis=1, keepdims=True)
        hit = col_s == t
        ax = jnp.where(hit, cx, ax)
        ay = jnp.where(hit, cy, ay)
        az = jnp.where(hit, cz, az)
        dx = X - cx
        dy = Y - cy
        dz = Z - cz
        d = dx * dx + dy * dy + dz * dz
        dist = jnp.minimum(dist, d)
        mx = jnp.max(dist, axis=1, keepdims=True)
        far_new = jnp.min(
            jnp.where(dist == mx, col, jnp.int32(n)), axis=1, keepdims=True
        )
        return dist, far_new, ax, ay, az

    init = (
        jnp.full((b, n), 1e10, jnp.float32),
        jnp.zeros((b, 1), jnp.int32),
        jnp.zeros((b, npoint), jnp.float32),
        jnp.zeros((b, npoint), jnp.float32),
        jnp.zeros((b, npoint), jnp.float32),
    )
    _, _, ax, ay, az = lax.fori_loop(0, npoint, step, init)
    nx_ref[...] = ax
    ny_ref[...] = ay
    nz_ref[...] = az


def _fps(X, Y, Z, npoint):
    b, n = X.shape
    out = jax.ShapeDtypeStruct((b, npoint), jnp.float32)
    nx, ny, nz = pl.pallas_call(
        functools.partial(_fps_body, npoint=npoint),
        out_shape=(out, out, out),
        interpret=_INTERPRET,
    )(X, Y, Z)
    return nx, ny, nz


def _mask_body(new_ref, ptsT_ref, o_ref, *, r2s):
    new = new_ref[0]
    ptsT = ptsT_ref[0]
    pn2 = jnp.sum(ptsT * ptsT, axis=0, keepdims=True)
    cn2 = jnp.sum(new * new, axis=1, keepdims=True)
    d = (cn2 + pn2) - 2.0 * jnp.dot(new, ptsT,
                                    preferred_element_type=jnp.float32)
    w = jnp.zeros(d.shape, jnp.int32)
    for i, r2 in enumerate(r2s):
        w = w + jnp.where(d <= jnp.float32(r2), jnp.int32(1 << i),
                          jnp.int32(0))
    o_ref[0] = w


def _masks(new_xyz, ptsT, r2s):
    b, s, _ = new_xyz.shape
    n = ptsT.shape[2]
    out = pl.pallas_call(
        functools.partial(_mask_body, r2s=tuple(r2s)),
        grid=(b,),
        in_specs=[
            pl.BlockSpec((1, s, 3), lambda i: (i, 0, 0)),
            pl.BlockSpec((1, 3, n), lambda i: (i, 0, 0)),
        ],
        out_specs=pl.BlockSpec((1, s, n), lambda i: (i, 0, 0)),
        out_shape=jax.ShapeDtypeStruct((b, s, n), jnp.int32),
        interpret=_INTERPRET,
    )(new_xyz, ptsT)
    return out.reshape(b * s, n)


def _table_body(*refs, nw):
    pts = refs[0][...]
    idx = 1
    xyzv = None
    if nw == 2:
        xyzv = refs[1][...]
        idx = 2
    n_br = (len(refs) - idx) // (nw + 2)
    ins = refs[idx:]
    outs = refs[idx + n_br * (nw + 1):]
    for br in range(n_br):
        wp = ins[br * (nw + 1)][...]
        t = jnp.dot(pts, wp, preferred_element_type=jnp.float32)
        if nw == 2:
            wx = ins[br * (nw + 1) + 1][...]
            t = t + jnp.dot(xyzv, wx, preferred_element_type=jnp.float32)
        bias = ins[br * (nw + 1) + nw][...]
        outs[br][...] = t + bias


def _tables(pts, xyz, weights):
    nw = 2 if xyz is not None else 1
    ins = [pts] + ([xyz] if xyz is not None else [])
    for wset in weights:
        ins.extend(wset)
    outs = tuple(
        jax.ShapeDtypeStruct((pts.shape[0], w[0].shape[1]), jnp.float32)
        for w in weights
    )
    return pl.pallas_call(
        functools.partial(_table_body, nw=nw),
        out_shape=outs,
        interpret=_INTERPRET,
    )(*ins)


@functools.lru_cache(maxsize=None)
def _sc_group_builder(BS, N, S, Ks, Cs):
    info = plsc.get_sparse_core_info()
    NC, NS = info.num_cores, info.num_subcores
    NW = NC * NS
    per_w = BS // NW
    nch = N // 16
    BN = (BS // S) * N

    out_type = tuple(
        jax.ShapeDtypeStruct((BS * K, C), jnp.float32)
        for K, C in zip(Ks, Cs)
    )
    scratch = [pltpu.VMEM((per_w * N,), jnp.int32)]
    for K, C in zip(Ks, Cs):
        scratch += [
            pltpu.VMEM((N + 16,), jnp.int32),
            pltpu.VMEM((2 * K,), jnp.int32),
            pltpu.VMEM((2 * K, C), jnp.float32),
        ]
    scratch += [pltpu.SemaphoreType.DMA] * 4
    mesh = plsc.VectorSubcoreMesh(core_axis_name="c", subcore_axis_name="s")

    @functools.partial(pl.kernel, mesh=mesh, out_type=out_type,
                       scratch_types=scratch,
                       compiler_params=pltpu.CompilerParams(
                           needs_layout_passes=False))
    def sc_kernel(mask_hbm, t0, t1, t2, g0, g1, g2,
                  mw, ib0, ix0, rw0, ib1, ix1, rw1, ib2, ix2, rw2,
                  semg0, semg1, semw0, semw1):
        T = (t0, t1, t2)
        G = (g0, g1, g2)
        IB = (ib0, ib1, ib2)
        IX = (ix0, ix1, ix2)
        RW = (rw0, rw1, rw2)
        SEMG = (semg0, semg1)
        SEMW = (semw0, semw1)
        wid = lax.axis_index("s") * NC + lax.axis_index("c")
        lane = lax.iota(jnp.int32, 16)
        zeros16 = jnp.zeros((16,), jnp.int32)
        ones16 = jnp.full((16,), 1, jnp.int32)

        def splat(x):
            return jnp.full((16,), x, jnp.int32)

        pmax16 = jnp.full((16,), N + 15, jnp.int32)
        imax16 = jnp.full((16,), BN - 1, jnp.int32)

        pltpu.sync_copy(mask_hbm.at[pl.ds(wid * per_w * N, per_w * N)], mw)

        def compact(ci, par):
            sg = wid * per_w + ci
            base = (sg // S) * N

            def chunk(ch, cnts):
                w = mw[pl.ds(ci * N + ch * 16, 16)]
                iv = lane + splat(ch * 16)
                new = []
                for br in range(3):
                    c = cnts[br]
                    m = ((w >> splat(br)) & ones16) == ones16
                    mi = m.astype(jnp.int32)
                    pos = plsc.cumsum(mi) + splat(c - 1)
                    pos = jnp.minimum(jnp.maximum(pos, zeros16), pmax16)
                    plsc.store_scatter(IB[br], [pos], iv, mask=m)
                    new.append(c + jnp.sum(mi))
                return tuple(new)

            cnts = lax.fori_loop(
                0, nch, chunk, (jnp.int32(0), jnp.int32(0), jnp.int32(0)))
            for br in range(3):
                K = Ks[br]
                c0 = IB[br][pl.ds(0, 16)]
                first = splat(jnp.sum(jnp.where(lane == zeros16, c0,
                                                zeros16)))
                cb = cnts[br]
                first = jnp.where(splat(cb) == zeros16,
                                  jnp.full((16,), N - 1, jnp.int32), first)
                for j in range(K // 16):
                    li = lane + splat(j * 16)
                    v = IB[br][pl.ds(j * 16, 16)]
                    v = jnp.where(li < splat(cb), v, first) + splat(base)
                    v = jnp.minimum(jnp.maximum(v, zeros16), imax16)
                    IX[br][pl.ds(par * K + j * 16, 16)] = v

        def fire_gathers(ci, par):
            for br in range(3):
                K = Ks[br]
                pltpu.async_copy(
                    T[br].at[IX[br].at[pl.ds(par * K, K)]],
                    RW[br].at[pl.ds(par * K, K)], SEMG[par])

        def drain_gathers(par):
            for br in range(3):
                K = Ks[br]
                pltpu.make_async_copy(
                    T[br].at[pl.ds(0, K)],
                    RW[br].at[pl.ds(par * K, K)], SEMG[par]).wait()

        def fire_writes(ci, par):
            sg = wid * per_w + ci
            for br in range(3):
                K = Ks[br]
                pltpu.async_copy(
                    RW[br].at[pl.ds(par * K, K)],
                    G[br].at[pl.ds(sg * K, K)], SEMW[par])

        def drain_writes(par):
            for br in range(3):
                K = Ks[br]
                pltpu.make_async_copy(
                    RW[br].at[pl.ds(par * K, K)],
                    G[br].at[pl.ds(0, K)], SEMW[par]).wait()

        compact(0, 0)
        fire_gathers(0, 0)
        compact(1, 1)
        fire_gathers(1, 1)

        def body(ip, _):
            for par in range(2):
                i = 2 * ip + par
                drain_gathers(par)
                fire_writes(i, par)
                nxt = jnp.minimum(i + 2, per_w - 1)
                compact(nxt, par)
                drain_writes(par)

                @pl.when(i + 2 < per_w)
                def _():
                    fire_gathers(i + 2, par)

            return 0

        lax.fori_loop(0, per_w // 2, body, 0)

    return sc_kernel


def _sc_group(mask, tables, BS, N, S, Ks, Cs):
    k = _sc_group_builder(BS, N, S, tuple(Ks), tuple(Cs))
    return k(mask.reshape(BS * N), *tables)


def _branch_body(g_ref, nxyz_ref, wx_ref, w2_ref, b2_ref, w3_ref, b3_ref,
                 o_ref, *, SB, K):
    D = jnp.dot(nxyz_ref[...], wx_ref[...],
                preferred_element_type=jnp.float32)
    G = g_ref[...]
    C1 = G.shape[1]
    h = jnp.maximum(G.reshape(SB, K, C1) - D[:, None, :], 0.0)
    h = h.reshape(SB * K, C1)
    h = jnp.maximum(
        jnp.dot(h, w2_ref[...], preferred_element_type=jnp.float32)
        + b2_ref[...], 0.0)
    h = jnp.maximum(
        jnp.dot(h, w3_ref[...], preferred_element_type=jnp.float32)
        + b3_ref[...], 0.0)
    C3 = h.shape[1]
    o_ref[...] = jnp.max(h.reshape(SB, K, C3), axis=1)


def _branch(G, nxyz, wx, w2, b2, w3, b3, K, SB):
    BS = nxyz.shape[0]
    C1 = wx.shape[1]
    C3 = w3.shape[1]
    grid = (BS // SB,)
    return pl.pallas_call(
        functools.partial(_branch_body, SB=SB, K=K),
        grid=grid,
        in_specs=[
            pl.BlockSpec((SB * K, C1), lambda i: (i, 0)),
            pl.BlockSpec((SB, 3), lambda i: (i, 0)),
            pl.BlockSpec(wx.shape, lambda i: (0, 0)),
            pl.BlockSpec(w2.shape, lambda i: (0, 0)),
            pl.BlockSpec(b2.shape, lambda i: (0, 0)),
            pl.BlockSpec(w3.shape, lambda i: (0, 0)),
            pl.BlockSpec(b3.shape, lambda i: (0, 0)),
        ],
        out_specs=pl.BlockSpec((SB, C3), lambda i: (i, 0)),
        out_shape=jax.ShapeDtypeStruct((BS, C3), jnp.float32),
        interpret=_INTERPRET,
    )(G, nxyz, wx, w2, b2, w3, b3)


def _final_body(xyz_ref, pts_ref, wx_ref, wp_ref, b1_ref, w2_ref, b2_ref,
                w3_ref, b3_ref, wf1_ref, bf1_ref, wf2_ref, bf2_ref,
                wf3_ref, bf3_ref, o_ref, *, b, s):
    h = jnp.dot(xyz_ref[...], wx_ref[...],
                preferred_element_type=jnp.float32)
    h = h + jnp.dot(pts_ref[...], wp_ref[...],
                    preferred_element_type=jnp.float32)
    h = jnp.maximum(h + b1_ref[...], 0.0)
    h = jnp.maximum(
        jnp.dot(h, w2_ref[...], preferred_element_type=jnp.float32)
        + b2_ref[...], 0.0)
    h = jnp.maximum(
        jnp.dot(h, w3_ref[...], preferred_element_type=jnp.float32)
        + b3_ref[...], 0.0)
    C = h.shape[1]
    hm = jnp.max(h.reshape(b, s, C), axis=1)
    f = jnp.maximum(
        jnp.dot(hm, wf1_ref[...], preferred_element_type=jnp.float32)
        + bf1_ref[...], 0.0)
    f = jnp.maximum(
        jnp.dot(f, wf2_ref[...], preferred_element_type=jnp.float32)
        + bf2_ref[...], 0.0)
    o_ref[...] = (
        jnp.dot(f, wf3_ref[...], preferred_element_type=jnp.float32)
        + bf3_ref[...])


def _final(xyz2, pts2, ws, b, s, ncls):
    return pl.pallas_call(
        functools.partial(_final_body, b=b, s=s),
        out_shape=jax.ShapeDtypeStruct((b, ncls), jnp.float32),
        interpret=_INTERPRET,
    )(xyz2, pts2, *ws)


def _fold(layer):
    w, bb, gamma, beta = layer
    s = gamma / jnp.sqrt(1.0 + _EPS)
    return (w * s[:, None]).T, (s * bb + beta)[None, :]


def _fold_split(layer, n_head):
    wf, bias = _fold(layer)
    return wf[:n_head], wf[n_head:], bias


_CP = 128


def _pad_cols(a):
    return jnp.pad(a, ((0, 0), (0, _CP - a.shape[1])))


_SA1 = dict(S=512, radii=(0.1, 0.2, 0.4), Ks=(16, 32, 128), SBs=(128, 128, 64))
_SA2 = dict(S=128, radii=(0.2, 0.4, 0.8), Ks=(32, 64, 128), SBs=(128, 64, 32))


def _sa_msg(pts_feat, X, Y, Z, cfg, branch_params, b, n):
    s = cfg["S"]
    nx, ny, nz = _fps(X, Y, Z, s)
    new_xyz = jnp.stack([nx, ny, nz], axis=-1)
    ptsT = jnp.stack([X, Y, Z], axis=1)
    mask = _masks(new_xyz, ptsT, [r * r for r in cfg["radii"]])

    xyz_flat = jnp.stack([X, Y, Z], axis=-1).reshape(b * n, 3)
    wsets = []
    folded = []
    for layers in branch_params:
        if pts_feat is None:
            wx, bias = _fold(layers[0])
            wx = _pad_cols(wx)
            bias = _pad_cols(bias)
            wsets.append((wx, bias))
            rest = [_fold(l) for l in layers[1:]]
        else:
            cin = pts_feat.shape[2]
            wp, wx, bias = _fold_split(layers[0], cin)
            wp, wx, bias = _pad_cols(wp), _pad_cols(wx), _pad_cols(bias)
            wsets.append((wp, wx, bias))
            rest = [_fold(l) for l in layers[1:]]
        (w2, b2), (w3, b3) = rest
        w2 = jnp.pad(w2, ((0, _CP - w2.shape[0]), (0, 0)))
        folded.append((wx, (w2, b2), (w3, b3)))

    if pts_feat is None:
        tabs = _tables(xyz_flat, None, wsets)
    else:
        tabs = _tables(pts_feat.reshape(b * n, -1), xyz_flat, wsets)

    Cs = tuple(t.shape[1] for t in tabs)
    Gs = _sc_group(mask, tabs, b * s, n, s, cfg["Ks"], Cs)

    new_flat = new_xyz.reshape(b * s, 3)
    outs = []
    for br in range(3):
        wx = folded[br][0]
        (w2, b2), (w3, b3) = folded[br][1], folded[br][2]
        o = _branch(Gs[br], new_flat, wx, w2, b2, w3, b3,
                    cfg["Ks"][br], cfg["SBs"][br])
        outs.append(o.reshape(b, s, -1))
    return new_xyz, jnp.concatenate(outs, axis=-1)


def kernel(xyz, params):
    b = xyz.shape[0]
    X, Y, Z = xyz[:, 0, :], xyz[:, 1, :], xyz[:, 2, :]

    l1_xyz, l1_pts = _sa_msg(None, X, Y, Z, _SA1, params["sa1"], b, _N)
    X2, Y2, Z2 = l1_xyz[..., 0], l1_xyz[..., 1], l1_xyz[..., 2]
    l2_xyz, l2_pts = _sa_msg(l1_pts, X2, Y2, Z2, _SA2, params["sa2"],
                             b, _SA1["S"])

    s2 = _SA2["S"]
    wx3, wp3, b31 = _fold_split(params["sa3"][0], 3)
    (w32, b32), (w33, b33) = (_fold(params["sa3"][1]),
                              _fold(params["sa3"][2]))
    wf1, bf1 = _fold(params["fc1"])
    wf2, bf2 = _fold(params["fc2"])
    w3, b3 = params["fc3"]
    ws = (wx3, wp3, b31, w32, b32, w33, b33,
          wf1, bf1, wf2, bf2, w3.T, b3[None, :])
    return _final(l2_xyz.reshape(b * s2, 3), l2_pts.reshape(b * s2, -1),
                  ws, b, s2, w3.shape[0])


# speedup vs baseline: 9.0370x; 1.0012x over previous
import functools

import jax
import jax.numpy as jnp
from jax import lax
from jax.experimental import pallas as pl
from jax.experimental.pallas import tpu as pltpu
from jax.experimental.pallas import tpu_sc as plsc

_EPS = 1e-5
_B, _N = 4, 1024
_INTERPRET = False


def _fps_body(x_ref, y_ref, z_ref, nx_ref, ny_ref, nz_ref, *, npoint):
    X = x_ref[...]
    Y = y_ref[...]
    Z = z_ref[...]
    b, n = X.shape
    col = lax.broadcasted_iota(jnp.int32, (b, n), 1)
    col_s = lax.broadcasted_iota(jnp.int32, (b, npoint), 1)

    def step(t, carry):
        dist, far, ax, ay, az = carry
        sel = col == far
        cx = jnp.sum(jnp.where(sel, X, 0.0), axis=1, keepdims=True)
        cy = jnp.sum(jnp.where(sel, Y, 0.0), axis=1, keepdims=True)
        cz = jnp.sum(jnp.where(sel, Z, 0.0), axis=1, keepdims=True)
        hit = col_s == t
        ax = jnp.where(hit, cx, ax)
        ay = jnp.where(hit, cy, ay)
        az = jnp.where(hit, cz, az)
        dx = X - cx
        dy = Y - cy
        dz = Z - cz
        d = dx * dx + dy * dy + dz * dz
        dist = jnp.minimum(dist, d)
        mx = jnp.max(dist, axis=1, keepdims=True)
        far_new = jnp.min(
            jnp.where(dist == mx, col, jnp.int32(n)), axis=1, keepdims=True
        )
        return dist, far_new, ax, ay, az

    init = (
        jnp.full((b, n), 1e10, jnp.float32),
        jnp.zeros((b, 1), jnp.int32),
        jnp.zeros((b, npoint), jnp.float32),
        jnp.zeros((b, npoint), jnp.float32),
        jnp.zeros((b, npoint), jnp.float32),
    )
    _, _, ax, ay, az = lax.fori_loop(0, npoint, step, init)
    nx_ref[...] = ax
    ny_ref[...] = ay
    nz_ref[...] = az


def _fps(X, Y, Z, npoint):
    b, n = X.shape
    out = jax.ShapeDtypeStruct((b, npoint), jnp.float32)
    nx, ny, nz = pl.pallas_call(
        functools.partial(_fps_body, npoint=npoint),
        out_shape=(out, out, out),
        interpret=_INTERPRET,
    )(X, Y, Z)
    return nx, ny, nz


def _mask_body(new_ref, ptsT_ref, o_ref, *, r2s):
    new = new_ref[0]
    ptsT = ptsT_ref[0]
    pn2 = jnp.sum(ptsT * ptsT, axis=0, keepdims=True)
    cn2 = jnp.sum(new * new, axis=1, keepdims=True)
    d = (cn2 + pn2) - 2.0 * jnp.dot(new, ptsT,
                                    preferred_element_type=jnp.float32)
    w = jnp.zeros(d.shape, jnp.int32)
    for i, r2 in enumerate(r2s):
        w = w + jnp.where(d <= jnp.float32(r2), jnp.int32(1 << i),
                          jnp.int32(0))
    o_ref[0] = w


def _masks(new_xyz, ptsT, r2s):
    b, s, _ = new_xyz.shape
    n = ptsT.shape[2]
    out = pl.pallas_call(
        functools.partial(_mask_body, r2s=tuple(r2s)),
        grid=(b,),
        in_specs=[
            pl.BlockSpec((1, s, 3), lambda i: (i, 0, 0)),
            pl.BlockSpec((1, 3, n), lambda i: (i, 0, 0)),
        ],
        out_specs=pl.BlockSpec((1, s, n), lambda i: (i, 0, 0)),
        out_shape=jax.ShapeDtypeStruct((b, s, n), jnp.int32),
        interpret=_INTERPRET,
    )(new_xyz, ptsT)
    return out.reshape(b * s, n)


def _table_body(*refs, nw):
    pts = refs[0][...]
    idx = 1
    xyzv = None
    if nw == 2:
        xyzv = refs[1][...]
        idx = 2
    n_br = (len(refs) - idx) // (nw + 2)
    ins = refs[idx:]
    outs = refs[idx + n_br * (nw + 1):]
    for br in range(n_br):
        wp = ins[br * (nw + 1)][...]
        t = jnp.dot(pts, wp, preferred_element_type=jnp.float32)
        if nw == 2:
            wx = ins[br * (nw + 1) + 1][...]
            t = t + jnp.dot(xyzv, wx, preferred_element_type=jnp.float32)
        bias = ins[br * (nw + 1) + nw][...]
        outs[br][...] = t + bias


def _tables(pts, xyz, weights):
    nw = 2 if xyz is not None else 1
    ins = [pts] + ([xyz] if xyz is not None else [])
    for wset in weights:
        ins.extend(wset)
    outs = tuple(
        jax.ShapeDtypeStruct((pts.shape[0], w[0].shape[1]), jnp.float32)
        for w in weights
    )
    return pl.pallas_call(
        functools.partial(_table_body, nw=nw),
        out_shape=outs,
        interpret=_INTERPRET,
    )(*ins)


@functools.lru_cache(maxsize=None)
def _sc_group_builder(BS, N, S, Ks, Cs):
    info = plsc.get_sparse_core_info()
    NC, NS = info.num_cores, info.num_subcores
    NW = NC * NS
    per_w = BS // NW
    nch = N // 16
    BN = (BS // S) * N

    out_type = tuple(
        jax.ShapeDtypeStruct((BS * K, C), jnp.float32)
        for K, C in zip(Ks, Cs)
    )
    scratch = [pltpu.VMEM((per_w * N,), jnp.int32)]
    for K in Ks:
        scratch += [
            pltpu.VMEM((N + 16,), jnp.int32),
            pltpu.VMEM((2 * K,), jnp.int32),
            pltpu.VMEM((2 * K, 128), jnp.float32),
        ]
    scratch += [pltpu.SemaphoreType.DMA] * 4
    mesh = plsc.VectorSubcoreMesh(core_axis_name="c", subcore_axis_name="s")

    @functools.partial(pl.kernel, mesh=mesh, out_type=out_type,
                       scratch_types=scratch,
                       compiler_params=pltpu.CompilerParams(
                           needs_layout_passes=False))
    def sc_kernel(mask_hbm, t0, t1, t2, g0, g1, g2,
                  mw, ib0, ix0, rw0, ib1, ix1, rw1, ib2, ix2, rw2,
                  semg0, semg1, semw0, semw1):
        T = (t0, t1, t2)
        G = (g0, g1, g2)
        IB = (ib0, ib1, ib2)
        IX = (ix0, ix1, ix2)
        RW = (rw0, rw1, rw2)
        SEMG = (semg0, semg1)
        SEMW = (semw0, semw1)
        wid = lax.axis_index("s") * NC + lax.axis_index("c")
        lane = lax.iota(jnp.int32, 16)
        zeros16 = jnp.zeros((16,), jnp.int32)
        ones16 = jnp.full((16,), 1, jnp.int32)

        def splat(x):
            return jnp.full((16,), x, jnp.int32)

        pmax16 = jnp.full((16,), N + 15, jnp.int32)
        imax16 = jnp.full((16,), BN - 1, jnp.int32)

        pltpu.sync_copy(mask_hbm.at[pl.ds(wid * per_w * N, per_w * N)], mw)

        def compact(ci, par):
            sg = wid * per_w + ci
            base = (sg // S) * N

            def chunk(ch, cnts):
                w = mw[pl.ds(ci * N + ch * 16, 16)]
                iv = lane + splat(ch * 16)
                new = []
                for br in range(3):
                    cv = cnts[br]
                    m = ((w >> splat(br)) & ones16) == ones16
                    mi = m.astype(jnp.int32)
                    pos = plsc.cumsum(mi) + cv - ones16
                    pos = jnp.minimum(jnp.maximum(pos, zeros16), pmax16)
                    plsc.store_scatter(IB[br], [pos], iv, mask=m)
                    new.append(cv + plsc.all_reduce_population_count(m))
                return tuple(new)

            cnts = lax.fori_loop(0, nch, chunk, (zeros16, zeros16, zeros16))
            for br in range(3):
                K = Ks[br]
                c0 = IB[br][pl.ds(0, 16)]
                first = splat(jnp.sum(jnp.where(lane == zeros16, c0,
                                                zeros16)))
                cb = cnts[br]
                first = jnp.where(cb == zeros16,
                                  jnp.full((16,), N - 1, jnp.int32), first)
                for j in range(K // 16):
                    li = lane + splat(j * 16)
                    v = IB[br][pl.ds(j * 16, 16)]
                    v = jnp.where(li < cb, v, first) + splat(base)
                    v = jnp.minimum(jnp.maximum(v, zeros16), imax16)
                    IX[br][pl.ds(par * K + j * 16, 16)] = v

        def fire_gathers(ci, par):
            for br in range(3):
                K = Ks[br]
                pltpu.async_copy(
                    T[br].at[IX[br].at[pl.ds(par * K, K)]],
                    RW[br].at[pl.ds(par * K, K)], SEMG[par])

        def drain_gathers(par):
            for br in range(3):
                K = Ks[br]
                pltpu.make_async_copy(
                    T[br].at[pl.ds(0, K)],
                    RW[br].at[pl.ds(par * K, K)], SEMG[par]).wait()

        def fire_writes(ci, par):
            sg = wid * per_w + ci
            for br in range(3):
                K = Ks[br]
                pltpu.async_copy(
                    RW[br].at[pl.ds(par * K, K)],
                    G[br].at[pl.ds(sg * K, K)], SEMW[par])

        def drain_writes(par):
            for br in range(3):
                K = Ks[br]
                pltpu.make_async_copy(
                    RW[br].at[pl.ds(par * K, K)],
                    G[br].at[pl.ds(0, K)], SEMW[par]).wait()

        compact(0, 0)
        fire_gathers(0, 0)
        compact(1, 1)
        fire_gathers(1, 1)

        def body(ip, _):
            for par in range(2):
                i = 2 * ip + par
                drain_gathers(par)
                fire_writes(i, par)
                nxt = jnp.minimum(i + 2, per_w - 1)
                compact(nxt, par)
                drain_writes(par)

                @pl.when(i + 2 < per_w)
                def _():
                    fire_gathers(i + 2, par)

            return 0

        lax.fori_loop(0, per_w // 2, body, 0)

    return sc_kernel


def _sc_group(mask, tables, BS, N, S, Ks, Cs):
    k = _sc_group_builder(BS, N, S, tuple(Ks), tuple(Cs))
    return k(mask.reshape(BS * N), *tables)


def _branch_body(g_ref, nxyz_ref, wx_ref, w2_ref, b2_ref, w3_ref, b3_ref,
                 o_ref, *, SB, K):
    D = jnp.dot(nxyz_ref[...], wx_ref[...],
                preferred_element_type=jnp.float32)
    G = g_ref[...]
    C1 = G.shape[1]
    h = jnp.maximum(G.reshape(SB, K, C1) - D[:, None, :], 0.0)
    h = h.reshape(SB * K, C1)
    h = jnp.maximum(
        jnp.dot(h, w2_ref[...], preferred_element_type=jnp.float32)
        + b2_ref[...], 0.0)
    h = jnp.maximum(
        jnp.dot(h, w3_ref[...], preferred_element_type=jnp.float32)
        + b3_ref[...], 0.0)
    C3 = h.shape[1]
    o_ref[...] = jnp.max(h.reshape(SB, K, C3), axis=1)


def _branch(G, nxyz, wx, w2, b2, w3, b3, K, SB):
    BS = nxyz.shape[0]
    C1 = wx.shape[1]
    C3 = w3.shape[1]
    grid = (BS // SB,)
    return pl.pallas_call(
        functools.partial(_branch_body, SB=SB, K=K),
        grid=grid,
        in_specs=[
            pl.BlockSpec((SB * K, C1), lambda i: (i, 0)),
            pl.BlockSpec((SB, 3), lambda i: (i, 0)),
            pl.BlockSpec(wx.shape, lambda i: (0, 0)),
            pl.BlockSpec(w2.shape, lambda i: (0, 0)),
            pl.BlockSpec(b2.shape, lambda i: (0, 0)),
            pl.BlockSpec(w3.shape, lambda i: (0, 0)),
            pl.BlockSpec(b3.shape, lambda i: (0, 0)),
        ],
        out_specs=pl.BlockSpec((SB, C3), lambda i: (i, 0)),
        out_shape=jax.ShapeDtypeStruct((BS, C3), jnp.float32),
        interpret=_INTERPRET,
    )(G, nxyz, wx, w2, b2, w3, b3)


def _final_body(xyz_ref, pts_ref, wx_ref, wp_ref, b1_ref, w2_ref, b2_ref,
                w3_ref, b3_ref, wf1_ref, bf1_ref, wf2_ref, bf2_ref,
                wf3_ref, bf3_ref, o_ref, *, b, s):
    h = jnp.dot(xyz_ref[...], wx_ref[...],
                preferred_element_type=jnp.float32)
    h = h + jnp.dot(pts_ref[...], wp_ref[...],
                    preferred_element_type=jnp.float32)
    h = jnp.maximum(h + b1_ref[...], 0.0)
    h = jnp.maximum(
        jnp.dot(h, w2_ref[...], preferred_element_type=jnp.float32)
        + b2_ref[...], 0.0)
    h = jnp.maximum(
        jnp.dot(h, w3_ref[...], preferred_element_type=jnp.float32)
        + b3_ref[...], 0.0)
    C = h.shape[1]
    hm = jnp.max(h.reshape(b, s, C), axis=1)
    f = jnp.maximum(
        jnp.dot(hm, wf1_ref[...], preferred_element_type=jnp.float32)
        + bf1_ref[...], 0.0)
    f = jnp.maximum(
        jnp.dot(f, wf2_ref[...], preferred_element_type=jnp.float32)
        + bf2_ref[...], 0.0)
    o_ref[...] = (
        jnp.dot(f, wf3_ref[...], preferred_element_type=jnp.float32)
        + bf3_ref[...])


def _final(xyz2, pts2, ws, b, s, ncls):
    return pl.pallas_call(
        functools.partial(_final_body, b=b, s=s),
        out_shape=jax.ShapeDtypeStruct((b, ncls), jnp.float32),
        interpret=_INTERPRET,
    )(xyz2, pts2, *ws)


def _fold(layer):
    w, bb, gamma, beta = layer
    s = gamma / jnp.sqrt(1.0 + _EPS)
    return (w * s[:, None]).T, (s * bb + beta)[None, :]


def _fold_split(layer, n_head):
    wf, bias = _fold(layer)
    return wf[:n_head], wf[n_head:], bias


_CP = 128


def _pad_cols(a):
    return jnp.pad(a, ((0, 0), (0, _CP - a.shape[1])))


_SA1 = dict(S=512, radii=(0.1, 0.2, 0.4), Ks=(16, 32, 128), SBs=(128, 128, 64))
_SA2 = dict(S=128, radii=(0.2, 0.4, 0.8), Ks=(32, 64, 128), SBs=(128, 64, 32))


def _sa_msg(pts_feat, X, Y, Z, cfg, branch_params, b, n):
    s = cfg["S"]
    nx, ny, nz = _fps(X, Y, Z, s)
    new_xyz = jnp.stack([nx, ny, nz], axis=-1)
    ptsT = jnp.stack([X, Y, Z], axis=1)
    mask = _masks(new_xyz, ptsT, [r * r for r in cfg["radii"]])

    xyz_flat = jnp.stack([X, Y, Z], axis=-1).reshape(b * n, 3)
    wsets = []
    folded = []
    for layers in branch_params:
        if pts_feat is None:
            wx, bias = _fold(layers[0])
            wsets.append((_pad_cols(wx), _pad_cols(bias)))
            rest = [_fold(l) for l in layers[1:]]
        else:
            cin = pts_feat.shape[2]
            wp, wx, bias = _fold_split(layers[0], cin)
            wsets.append((_pad_cols(wp), _pad_cols(wx), _pad_cols(bias)))
            rest = [_fold(l) for l in layers[1:]]
        (w2, b2), (w3, b3) = rest
        w2 = jnp.pad(w2, ((0, _CP - w2.shape[0]), (0, 0)))
        folded.append((_pad_cols(wx), (w2, b2), (w3, b3)))

    if pts_feat is None:
        tabs = _tables(xyz_flat, None, wsets)
    else:
        tabs = _tables(pts_feat.reshape(b * n, -1), xyz_flat, wsets)

    Cs = (_CP, _CP, _CP)
    Gs = _sc_group(mask, tabs, b * s, n, s, cfg["Ks"], Cs)

    new_flat = new_xyz.reshape(b * s, 3)
    outs = []
    for br in range(3):
        wx = folded[br][0]
        (w2, b2), (w3, b3) = folded[br][1], folded[br][2]
        o = _branch(Gs[br], new_flat, wx, w2, b2, w3, b3,
                    cfg["Ks"][br], cfg["SBs"][br])
        outs.append(o.reshape(b, s, -1))
    return new_xyz, jnp.concatenate(outs, axis=-1)


def kernel(xyz, params):
    b = xyz.shape[0]
    X, Y, Z = xyz[:, 0, :], xyz[:, 1, :], xyz[:, 2, :]

    l1_xyz, l1_pts = _sa_msg(None, X, Y, Z, _SA1, params["sa1"], b, _N)
    X2, Y2, Z2 = l1_xyz[..., 0], l1_xyz[..., 1], l1_xyz[..., 2]
    l2_xyz, l2_pts = _sa_msg(l1_pts, X2, Y2, Z2, _SA2, params["sa2"],
                             b, _SA1["S"])

    s2 = _SA2["S"]
    wx3, wp3, b31 = _fold_split(params["sa3"][0], 3)
    (w32, b32), (w33, b33) = (_fold(params["sa3"][1]),
                              _fold(params["sa3"][2]))
    wf1, bf1 = _fold(params["fc1"])
    wf2, bf2 = _fold(params["fc2"])
    w3, b3 = params["fc3"]
    ws = (wx3, wp3, b31, w32, b32, w33, b33,
          wf1, bf1, wf2, bf2, w3.T, b3[None, :])
    return _final(l2_xyz.reshape(b * s2, 3), l2_pts.reshape(b * s2, -1),
                  ws, b, s2, w3.shape[0])
